# Initial kernel scaffold; baseline (speedup 1.0000x reference)
import functools

import jax
import jax.numpy as jnp
from jax import lax
from jax.experimental import pallas as pl
from jax.experimental.pallas import tpu as pltpu
from jax.experimental.pallas import tpu_sc as plsc

NC = 2
NS = 16
NW = NC * NS
CHUNK = 128
NBUF = 8


def _make_embed(B, D, K):
    mesh = plsc.VectorSubcoreMesh(core_axis_name="c", subcore_axis_name="s")
    sems = [pltpu.SemaphoreType.DMA for _ in range(NBUF)]

    @functools.partial(
        pl.kernel,
        out_type=jax.ShapeDtypeStruct((B, D), jnp.float32),
        mesh=mesh,
        scratch_types=[
            pltpu.VMEM((K, CHUNK), jnp.int32),
            pltpu.VMEM((NBUF, CHUNK, D), jnp.float32),
        ] + sems,
    )
    def embed(idx_hbm, table_hbm, out_hbm, idx_v, rows_v, *gsem):
        wid = lax.axis_index("s") * NC + lax.axis_index("c")
        base = wid * (K * CHUNK)

        pltpu.sync_copy(idx_hbm.at[wid], idx_v)

        def start_gather(j, b):
            pltpu.async_copy(table_hbm.at[idx_v.at[j]], rows_v.at[b], gsem[b])

        def wait_gather(j, b):
            pltpu.make_async_copy(
                table_hbm.at[idx_v.at[j]], rows_v.at[b], gsem[b]
            ).wait()

        def write_out(j, b):
            pltpu.sync_copy(rows_v.at[b], out_hbm.at[pl.ds(base + j * CHUNK, CHUNK)])

        for b in range(NBUF):
            start_gather(b, b)

        def block(i, carry):
            j0 = i * NBUF
            for b in range(NBUF):
                wait_gather(j0 + b, b)
                write_out(j0 + b, b)
                start_gather(j0 + b + NBUF, b)
            return carry

        lax.fori_loop(0, (K - NBUF) // NBUF, block, 0)

        for b in range(NBUF):
            j = K - NBUF + b
            wait_gather(j, b)
            write_out(j, b)

    return embed


def kernel(x, weight):
    S0, S1 = x.shape
    B = S0 * S1
    D = weight.shape[1]
    assert B % (NW * CHUNK) == 0
    K = B // (NW * CHUNK)
    idx = x.reshape(NW, K, CHUNK).astype(jnp.int32)
    out = _make_embed(B, D, K)(idx, weight)
    return out.reshape(S0, S1, D)


# speedup vs baseline: 1.1124x; 1.1124x over previous
import functools

import jax
import jax.numpy as jnp
from jax import lax
from jax.experimental import pallas as pl
from jax.experimental.pallas import tpu as pltpu
from jax.experimental.pallas import tpu_sc as plsc

NC = 2
NS = 16
NW = NC * NS
CHUNK = 128
NBUF = 8


def _make_embed(B, D, K):
    mesh = plsc.VectorSubcoreMesh(core_axis_name="c", subcore_axis_name="s")
    sems = [pltpu.SemaphoreType.DMA for _ in range(NBUF)]

    @functools.partial(
        pl.kernel,
        out_type=jax.ShapeDtypeStruct((B, D), jnp.float32),
        mesh=mesh,
        scratch_types=[
            pltpu.VMEM((K, CHUNK), jnp.int32),
            pltpu.VMEM((NBUF, CHUNK, D), jnp.float32),
        ] + sems,
        compiler_params=pltpu.CompilerParams(use_tc_tiling_on_sc=False),
    )
    def embed(idx_hbm, table_hbm, out_hbm, idx_v, rows_v, *gsem):
        wid = lax.axis_index("s") * NC + lax.axis_index("c")
        base = wid * (K * CHUNK)

        pltpu.sync_copy(idx_hbm.at[wid], idx_v)

        def start_gather(j, b):
            pltpu.async_copy(table_hbm.at[idx_v.at[j]], rows_v.at[b], gsem[b])

        def wait_gather(j, b):
            pltpu.make_async_copy(
                table_hbm.at[idx_v.at[j]], rows_v.at[b], gsem[b]
            ).wait()

        def write_out(j, b):
            pltpu.sync_copy(rows_v.at[b], out_hbm.at[pl.ds(base + j * CHUNK, CHUNK)])

        for b in range(NBUF):
            start_gather(b, b)

        def block(i, carry):
            j0 = i * NBUF
            for b in range(NBUF):
                wait_gather(j0 + b, b)
                write_out(j0 + b, b)
                start_gather(j0 + b + NBUF, b)
            return carry

        lax.fori_loop(0, (K - NBUF) // NBUF, block, 0)

        for b in range(NBUF):
            j = K - NBUF + b
            wait_gather(j, b)
            write_out(j, b)

    return embed


def kernel(x, weight):
    S0, S1 = x.shape
    B = S0 * S1
    D = weight.shape[1]
    assert B % (NW * CHUNK) == 0
    K = B // (NW * CHUNK)
    idx = x.reshape(NW, K, CHUNK).astype(jnp.int32)
    out = _make_embed(B, D, K)(idx, weight)
    return out.reshape(S0, S1, D)


# speedup vs baseline: 1.1126x; 1.0001x over previous
import functools

import jax
import jax.numpy as jnp
from jax import lax
from jax.experimental import pallas as pl
from jax.experimental.pallas import tpu as pltpu
from jax.experimental.pallas import tpu_sc as plsc

NC = 2
NS = 16
NW = NC * NS
CHUNK = 128
DEPTH = 8
NROWBUF = 2 * DEPTH


def _make_embed(B, D, K):
    assert K >= 2 * DEPTH and (K - 2 * DEPTH) % NROWBUF == 0
    mesh = plsc.VectorSubcoreMesh(core_axis_name="c", subcore_axis_name="s")
    sems = [pltpu.SemaphoreType.DMA for _ in range(NROWBUF)]

    @functools.partial(
        pl.kernel,
        out_type=jax.ShapeDtypeStruct((B, D), jnp.float32),
        mesh=mesh,
        scratch_types=[
            pltpu.VMEM((K, CHUNK), jnp.int32),
            pltpu.VMEM((NROWBUF, CHUNK, D), jnp.float32),
        ] + sems,
        compiler_params=pltpu.CompilerParams(use_tc_tiling_on_sc=False),
    )
    def embed(idx_hbm, table_hbm, out_hbm, idx_v, rows_v, *sem):
        wid = lax.axis_index("s") * NC + lax.axis_index("c")
        base = wid * (K * CHUNK)

        pltpu.sync_copy(idx_hbm.at[wid], idx_v)

        def start_gather(j, b):
            pltpu.async_copy(table_hbm.at[idx_v.at[j]], rows_v.at[b], sem[b])

        def wait_gather(j, b):
            pltpu.make_async_copy(
                table_hbm.at[idx_v.at[j]], rows_v.at[b], sem[b]
            ).wait()

        def start_write(j, b):
            pltpu.async_copy(
                rows_v.at[b], out_hbm.at[pl.ds(base + j * CHUNK, CHUNK)], sem[b]
            )

        def wait_write(j, b):
            pltpu.make_async_copy(
                rows_v.at[b], out_hbm.at[pl.ds(base + j * CHUNK, CHUNK)], sem[b]
            ).wait()

        for j in range(DEPTH):
            start_gather(j, j)

        for j in range(DEPTH):
            wait_gather(j, j)
            start_write(j, j)
            start_gather(j + DEPTH, j + DEPTH)

        def block(i, carry):
            j0 = DEPTH + i * NROWBUF
            for t in range(NROWBUF):
                j = j0 + t
                bg = (DEPTH + t) % NROWBUF
                bn = t
                wait_gather(j, bg)
                start_write(j, bg)
                wait_write(j - DEPTH, bn)
                start_gather(j + DEPTH, bn)
            return carry

        lax.fori_loop(0, (K - 2 * DEPTH) // NROWBUF, block, 0)

        for j in range(K - DEPTH, K):
            wait_gather(j, j % NROWBUF)
            start_write(j, j % NROWBUF)
        for j in range(K - NROWBUF, K):
            wait_write(j, j % NROWBUF)

    return embed


def kernel(x, weight):
    S0, S1 = x.shape
    B = S0 * S1
    D = weight.shape[1]
    assert B % (NW * CHUNK) == 0
    K = B // (NW * CHUNK)
    idx = x.reshape(NW, K, CHUNK).astype(jnp.int32)
    out = _make_embed(B, D, K)(idx, weight)
    return out.reshape(S0, S1, D)


# speedup vs baseline: 4.4329x; 3.9843x over previous
import functools

import jax
import jax.numpy as jnp
from jax import lax
from jax.experimental import pallas as pl
from jax.experimental.pallas import tpu as pltpu
from jax.experimental.pallas import tpu_sc as plsc

NC = 2
NS = 16
NW = NC * NS
DEPTH = 8
NROWBUF = 2 * DEPTH


def _make_embed(S0, S1, D):
    assert S0 % NW == 0
    K = S0 // NW
    assert K >= 2 * DEPTH and (K - 2 * DEPTH) % NROWBUF == 0
    mesh = plsc.VectorSubcoreMesh(core_axis_name="c", subcore_axis_name="s")
    sems = [pltpu.SemaphoreType.DMA for _ in range(NROWBUF)]

    @functools.partial(
        pl.kernel,
        out_type=jax.ShapeDtypeStruct((S0, S1, D), jnp.float32),
        mesh=mesh,
        scratch_types=[
            pltpu.VMEM((K, S1), jnp.int32),
            pltpu.VMEM((NROWBUF, S1, D), jnp.float32),
        ] + sems,
        compiler_params=pltpu.CompilerParams(use_tc_tiling_on_sc=False),
    )
    def embed(x_hbm, table_hbm, out_hbm, idx_v, rows_v, *sem):
        wid = lax.axis_index("s") * NC + lax.axis_index("c")
        i0 = wid * K

        pltpu.sync_copy(x_hbm.at[pl.ds(i0, K)], idx_v)

        def start_gather(r, b):
            pltpu.async_copy(table_hbm.at[idx_v.at[r]], rows_v.at[b], sem[b])

        def wait_gather(r, b):
            pltpu.make_async_copy(
                table_hbm.at[idx_v.at[r]], rows_v.at[b], sem[b]
            ).wait()

        def start_write(r, b):
            pltpu.async_copy(rows_v.at[b], out_hbm.at[i0 + r], sem[b])

        def wait_write(r, b):
            pltpu.make_async_copy(
                rows_v.at[b], out_hbm.at[i0 + r], sem[b]
            ).wait()

        for r in range(DEPTH):
            start_gather(r, r)

        for r in range(DEPTH):
            wait_gather(r, r)
            start_write(r, r)
            start_gather(r + DEPTH, r + DEPTH)

        def block(i, carry):
            r0 = DEPTH + i * NROWBUF
            for t in range(NROWBUF):
                r = r0 + t
                bg = (DEPTH + t) % NROWBUF
                bn = t
                wait_gather(r, bg)
                start_write(r, bg)
                wait_write(r - DEPTH, bn)
                start_gather(r + DEPTH, bn)
            return carry

        lax.fori_loop(0, (K - 2 * DEPTH) // NROWBUF, block, 0)

        for r in range(K - DEPTH, K):
            wait_gather(r, r % NROWBUF)
            start_write(r, r % NROWBUF)
        for r in range(K - NROWBUF, K):
            wait_write(r, r % NROWBUF)

    return embed


def kernel(x, weight):
    S0, S1 = x.shape
    D = weight.shape[1]
    return _make_embed(S0, S1, D)(x.astype(jnp.int32), weight)


# speedup vs baseline: 4.6908x; 1.0582x over previous
import functools

import jax
import jax.numpy as jnp
from jax import lax
from jax.experimental import pallas as pl
from jax.experimental.pallas import tpu as pltpu
from jax.experimental.pallas import tpu_sc as plsc

NC = 2
NS = 16
NW = NC * NS
DEPTH = 8
NROWBUF = 2 * DEPTH


def _make_embed(S0, S1, D):
    assert S0 % NW == 0
    K = S0 // NW
    assert K >= 2 * DEPTH and (K - 2 * DEPTH) % NROWBUF == 0
    mesh = plsc.VectorSubcoreMesh(core_axis_name="c", subcore_axis_name="s")
    sems = [pltpu.SemaphoreType.DMA for _ in range(NROWBUF)]

    @functools.partial(
        pl.kernel,
        out_type=jax.ShapeDtypeStruct((S0, S1, D), jnp.float32),
        mesh=mesh,
        scratch_types=[
            pltpu.VMEM((K, S1), jnp.int32),
            pltpu.VMEM((NROWBUF, S1, D), jnp.float32),
        ] + sems,
        compiler_params=pltpu.CompilerParams(use_tc_tiling_on_sc=False),
    )
    def embed(x_hbm, table_hbm, out_hbm, idx_v, rows_v, *sem):
        wid = lax.axis_index("s") * NC + lax.axis_index("c")
        i0 = wid * K

        pltpu.sync_copy(x_hbm.at[pl.ds(i0, K)], idx_v)

        def start_gather(r, b):
            pltpu.async_copy(table_hbm.at[idx_v.at[r]], rows_v.at[b], sem[b])

        def wait_gather(r, b):
            pltpu.make_async_copy(
                table_hbm.at[idx_v.at[r]], rows_v.at[b], sem[b]
            ).wait()

        def start_write(r, b):
            pltpu.async_copy(rows_v.at[b], out_hbm.at[i0 + r], sem[b])

        def wait_write(r, b):
            pltpu.make_async_copy(
                rows_v.at[b], out_hbm.at[i0 + r], sem[b]
            ).wait()

        for r in range(DEPTH):
            start_gather(r, r)

        for r in range(DEPTH):
            wait_gather(r, r)
            start_write(r, r)
            start_gather(r + DEPTH, r + DEPTH)

        def block(i, carry):
            r0 = DEPTH + i * NROWBUF
            for t in range(NROWBUF):
                r = r0 + t
                bg = (DEPTH + t) % NROWBUF
                bn = t
                wait_gather(r, bg)
                start_write(r, bg)
                wait_write(r - DEPTH, bn)
                start_gather(r + DEPTH, bn)
            return carry

        lax.fori_loop(0, (K - 2 * DEPTH) // NROWBUF, block, 0)

        for r in range(K - DEPTH, K):
            wait_gather(r, r % NROWBUF)
            start_write(r, r % NROWBUF)
        for r in range(K - NROWBUF, K):
            wait_write(r, r % NROWBUF)

    return embed


NCHUNK = 4


def kernel(x, weight):
    S0, S1 = x.shape
    D = weight.shape[1]
    xi = x.astype(jnp.int32)
    CS = S0 // NCHUNK
    embed = _make_embed(CS, S1, D)
    parts = [embed(xi[c * CS:(c + 1) * CS], weight) for c in range(NCHUNK)]
    return jnp.concatenate(parts, axis=0)
